# baseline (speedup 1.0000x reference)
import dataclasses
import functools

import jax
import jax.numpy as jnp
from jax import lax
from jax.experimental import pallas as pl
from jax.experimental.pallas import tpu as pltpu
from jax.experimental.pallas import tpu_sc as plsc

N_NODES = 10000
D = 256
E = 160000
NC = 2
NS = 16
NW = NC * NS
B_PER_W = E // NW
W = 200
NCHUNK = B_PER_W // W
LANES = 16
BLANES = 32
OUTP = W + (-W) % LANES
IDX_PER_W = 2 * B_PER_W


def _dot_kernel(table_hbm, ei_hbm, out_hbm,
                idx_v, buf0, buf1, outv0, outv1,
                sg0, sg1, so0, so1):
    wid = lax.axis_index("s") * NC + lax.axis_index("c")
    base = wid * B_PER_W
    pltpu.sync_copy(ei_hbm.at[pl.ds(wid * IDX_PER_W, IDX_PER_W)], idx_v)

    lane = lax.iota(jnp.int32, LANES)
    last_lane = jnp.full((LANES,), LANES - 1, jnp.int32)

    def _edge_dot(buf, w):
        pair = 2
        acc_lo = acc_hi = None
        for c0 in range(0, D // BLANES, pair):
            psum = None
            for c in range(c0, c0 + pair):
                av = plsc.bitcast(buf[w, pl.ds(c * LANES, LANES)],
                                  jnp.bfloat16)
                bv = plsc.bitcast(buf[W + w, pl.ds(c * LANES, LANES)],
                                  jnp.bfloat16)
                prod = av * bv
                psum = prod if psum is None else psum + prod
            lo, hi = plsc.unpack(psum, format=plsc.PackFormat.INTERLEAVED)
            acc_lo = lo if acc_lo is None else acc_lo + lo
            acc_hi = hi if acc_hi is None else acc_hi + hi
        cs = jnp.cumsum(acc_lo + acc_hi)
        return lax.gather(
            cs, last_lane[:, None],
            lax.GatherDimensionNumbers(offset_dims=(),
                                       collapsed_slice_dims=(0,),
                                       start_index_map=(0,)),
            slice_sizes=(1,),
            mode=lax.GatherScatterMode.PROMISE_IN_BOUNDS)

    def _issue(k, buf, sem_g):
        return pltpu.async_copy(
            table_hbm.at[idx_v.at[pl.ds(k * 2 * W, 2 * W)]], buf, sem_g)

    def _group_of(buf, w0, n_edges):
        res = jnp.zeros((LANES,), jnp.float32)
        for j in range(n_edges):
            res = jnp.where(lane == j, _edge_dot(buf, w0 + j), res)
        return res

    def _wait_out(outv, sem_o):
        pltpu.make_async_copy(outv.at[pl.ds(0, W)],
                              out_hbm.at[pl.ds(base, W)], sem_o).wait()

    def _compute_resident(k, buf, outv, sem_o, wait_prev):
        if wait_prev is not None:
            @pl.when(wait_prev)
            def _():
                _wait_out(outv, sem_o)

        @pl.loop(0, W // LANES)
        def _group(g):
            outv[pl.ds(g * LANES, LANES)] = _group_of(buf, g * LANES, LANES)

        if W % LANES:
            outv[pl.ds((W // LANES) * LANES, LANES)] = _group_of(
                buf, (W // LANES) * LANES, W % LANES)

        pltpu.async_copy(outv.at[pl.ds(0, W)],
                         out_hbm.at[pl.ds(base + k * W, W)], sem_o)

    _issue(0, buf0, sg0).wait()

    @pl.loop(0, NCHUNK - 1, step=2)
    def _pair(k):
        cp = _issue(k + 1, buf1, sg1)
        _compute_resident(k, buf0, outv0, so0, k > 0)
        cp.wait()
        cp2 = _issue(k + 2, buf0, sg0)
        _compute_resident(k + 1, buf1, outv1, so1, k > 0)
        cp2.wait()

    _wait_out(outv0, so0)
    _compute_resident(NCHUNK - 1, buf0, outv0, so0, None)
    _wait_out(outv0, so0)
    _wait_out(outv1, so1)


@jax.jit
def kernel(node_representations, edge_index):
    ei = (edge_index.astype(jnp.int32)
          .reshape(2, NW, NCHUNK, W)
          .transpose(1, 2, 0, 3)
          .reshape(2 * E))
    lo = lax.bitcast_convert_type(
        node_representations[:, :D // 2].astype(jnp.bfloat16),
        jnp.uint16).astype(jnp.uint32)
    hi = lax.bitcast_convert_type(
        node_representations[:, D // 2:].astype(jnp.bfloat16),
        jnp.uint16).astype(jnp.uint32)
    table = lax.bitcast_convert_type(lo | (hi << 16), jnp.int32)

    mesh = plsc.VectorSubcoreMesh(core_axis_name="c", subcore_axis_name="s")
    cp = pltpu.CompilerParams()
    if "needs_layout_passes" in pltpu.CompilerParams.__dataclass_fields__:
        cp = dataclasses.replace(cp, needs_layout_passes=False)
    k = functools.partial(
        pl.kernel,
        mesh=mesh,
        compiler_params=cp,
        out_type=jax.ShapeDtypeStruct((E,), jnp.float32),
        scratch_types=[
            pltpu.VMEM((IDX_PER_W,), jnp.int32),
            pltpu.VMEM((2 * W, D // 2), jnp.int32),
            pltpu.VMEM((2 * W, D // 2), jnp.int32),
            pltpu.VMEM((OUTP,), jnp.float32),
            pltpu.VMEM((OUTP,), jnp.float32),
            pltpu.SemaphoreType.DMA,
            pltpu.SemaphoreType.DMA,
            pltpu.SemaphoreType.DMA,
            pltpu.SemaphoreType.DMA,
        ],
    )(_dot_kernel)
    logits = k(table, ei)
    return logits.reshape(E, 1)
